# baseline (speedup 1.0000x reference)
import jax
import jax.numpy as jnp
from jax import lax
from jax.experimental import pallas as pl
from jax.experimental.pallas import tpu as pltpu
from jax.experimental.pallas import tpu_sc as plsc

ROWS = 256
PER_ROW = 3 * 224 * 224
SUB = PER_ROW // 128
LANE = 128
CHUNKS = ROWS // 16


def _compact_body(y_hbm, perm_hbm, n_hbm, y_v, perm_v, n_v):
    cid = lax.axis_index("c")
    sid = lax.axis_index("s")

    @pl.when(jnp.logical_and(cid == 0, sid == 0))
    def _():
        pltpu.sync_copy(y_hbm, y_v)
        lane = lax.iota(jnp.int32, 16)
        last = jnp.full((16,), 15, jnp.int32)
        zero = jnp.zeros((16,), jnp.int32)
        one = jnp.full((16,), 1, jnp.int32)
        for i in range(CHUNKS):
            perm_v[pl.ds(i * 16, 16)] = zero
        base = zero
        for i in range(CHUNKS):
            yv = y_v[pl.ds(i * 16, 16)]
            m = yv == one
            mi = jnp.where(m, one, zero)
            c = plsc.cumsum(mi)
            pos = base + c - mi
            plsc.store_scatter(perm_v, [pos], lane + (i * 16), mask=m)
            base = base + lax.gather(
                c,
                last[:, None],
                lax.GatherDimensionNumbers(
                    offset_dims=(),
                    collapsed_slice_dims=(0,),
                    start_index_map=(0,),
                ),
                slice_sizes=(1,),
                mode=lax.GatherScatterMode.PROMISE_IN_BOUNDS,
            )
        n_v[...] = base
        pltpu.sync_copy(perm_v, perm_hbm)
        pltpu.sync_copy(n_v, n_hbm)


_COMPACT_CACHE = []


def _compact(y):
    if not _COMPACT_CACHE:
        _COMPACT_CACHE.append(
            pl.kernel(
                _compact_body,
                out_type=(
                    jax.ShapeDtypeStruct((ROWS,), jnp.int32),
                    jax.ShapeDtypeStruct((16,), jnp.int32),
                ),
                mesh=plsc.VectorSubcoreMesh(
                    core_axis_name="c", subcore_axis_name="s"
                ),
                compiler_params=pltpu.CompilerParams(needs_layout_passes=False),
                scratch_types=[
                    pltpu.VMEM((ROWS,), jnp.int32),
                    pltpu.VMEM((ROWS,), jnp.int32),
                    pltpu.VMEM((16,), jnp.int32),
                ],
            )
        )
    return _COMPACT_CACHE[0](y)


def _loss_body(perm_ref, n_ref, r_ref, x_ref, out_ref):
    i = pl.program_id(0)
    n = n_ref[0]

    @pl.when(i == 0)
    def _():
        out_ref[0, 0] = jnp.float32(0.0)

    @pl.when(i < n)
    def _():
        out_ref[0, 0] += jnp.sum(jnp.abs(r_ref[...] - x_ref[...]))

    @pl.when(i == ROWS - 1)
    def _():
        total = out_ref[0, 0]
        denom = n.astype(jnp.float32) * jnp.float32(PER_ROW)
        out_ref[0, 0] = jnp.where(n > 0, total / denom, jnp.float32(0.0))


def _row_spec():
    return pl.BlockSpec(
        (1, 3, 224, 224),
        lambda i, perm, nv: (perm[jnp.minimum(i, jnp.maximum(nv[0], 1) - 1)], 0, 0, 0),
    )


_loss = pl.pallas_call(
    _loss_body,
    grid_spec=pltpu.PrefetchScalarGridSpec(
        num_scalar_prefetch=2,
        grid=(ROWS,),
        in_specs=[_row_spec(), _row_spec()],
        out_specs=pl.BlockSpec(memory_space=pltpu.SMEM),
    ),
    out_shape=jax.ShapeDtypeStruct((1, 1), jnp.float32),
)


def kernel(recons, x, y):
    perm, nvec = _compact(y)
    out = _loss(perm, nvec, recons, x)
    return out[0, 0]


# speedup vs baseline: 1.3518x; 1.3518x over previous
import jax
import jax.numpy as jnp
from jax import lax
from jax.experimental import pallas as pl
from jax.experimental.pallas import tpu as pltpu
from jax.experimental.pallas import tpu_sc as plsc

ROWS = 256
PER_ROW = 3 * 224 * 224
SUB = PER_ROW // 128
LANE = 128
CHUNKS = ROWS // 16


def _compact_body(y_hbm, perm_hbm, n_hbm, y_v, perm_v, n_v):
    cid = lax.axis_index("c")
    sid = lax.axis_index("s")

    @pl.when(jnp.logical_and(cid == 0, sid == 0))
    def _():
        pltpu.sync_copy(y_hbm, y_v)
        lane = lax.iota(jnp.int32, 16)
        last = jnp.full((16,), 15, jnp.int32)
        zero = jnp.zeros((16,), jnp.int32)
        one = jnp.full((16,), 1, jnp.int32)
        for i in range(CHUNKS):
            perm_v[pl.ds(i * 16, 16)] = zero
        base = zero
        for i in range(CHUNKS):
            yv = y_v[pl.ds(i * 16, 16)]
            m = yv == one
            mi = jnp.where(m, one, zero)
            c = plsc.cumsum(mi)
            pos = base + c - mi
            plsc.store_scatter(perm_v, [pos], lane + (i * 16), mask=m)
            base = base + lax.gather(
                c,
                last[:, None],
                lax.GatherDimensionNumbers(
                    offset_dims=(),
                    collapsed_slice_dims=(0,),
                    start_index_map=(0,),
                ),
                slice_sizes=(1,),
                mode=lax.GatherScatterMode.PROMISE_IN_BOUNDS,
            )
        n_v[...] = base
        pltpu.sync_copy(perm_v, perm_hbm)
        pltpu.sync_copy(n_v, n_hbm)


_COMPACT_CACHE = []


def _compact(y):
    if not _COMPACT_CACHE:
        _COMPACT_CACHE.append(
            pl.kernel(
                _compact_body,
                out_type=(
                    jax.ShapeDtypeStruct((ROWS,), jnp.int32),
                    jax.ShapeDtypeStruct((16,), jnp.int32),
                ),
                mesh=plsc.VectorSubcoreMesh(
                    core_axis_name="c", subcore_axis_name="s"
                ),
                compiler_params=pltpu.CompilerParams(needs_layout_passes=False),
                scratch_types=[
                    pltpu.VMEM((ROWS,), jnp.int32),
                    pltpu.VMEM((ROWS,), jnp.int32),
                    pltpu.VMEM((16,), jnp.int32),
                ],
            )
        )
    return _COMPACT_CACHE[0](y)


NBUF = 8
ROW_SUB = SUB


def _loss_body(perm_ref, n_ref, r_hbm, x_hbm, out_ref, rbuf, xbuf, acc, sems):
    n = n_ref[0]

    def start(k):
        slot = lax.rem(k, NBUF)
        row = perm_ref[k]
        pltpu.make_async_copy(r_hbm.at[row], rbuf.at[slot], sems.at[0, slot]).start()
        pltpu.make_async_copy(x_hbm.at[row], xbuf.at[slot], sems.at[1, slot]).start()

    def prime(k, carry):
        @pl.when(k < n)
        def _():
            start(k)

        return carry

    lax.fori_loop(0, NBUF, prime, 0)
    acc[...] = jnp.zeros((8, LANE), jnp.float32)

    def step(k, carry):
        slot = lax.rem(k, NBUF)
        row = perm_ref[k]
        pltpu.make_async_copy(r_hbm.at[row], rbuf.at[slot], sems.at[0, slot]).wait()
        pltpu.make_async_copy(x_hbm.at[row], xbuf.at[slot], sems.at[1, slot]).wait()
        d = jnp.abs(rbuf[slot] - xbuf[slot])
        acc[...] += jnp.sum(d.reshape(ROW_SUB // 8, 8, LANE), axis=0)

        @pl.when(k + NBUF < n)
        def _():
            start(k + NBUF)

        return carry

    lax.fori_loop(0, n, step, 0)
    total = jnp.sum(acc[...])
    denom = n.astype(jnp.float32) * jnp.float32(PER_ROW)
    out_ref[0, 0] = jnp.where(n > 0, total / denom, jnp.float32(0.0))


_loss = pl.pallas_call(
    _loss_body,
    grid_spec=pltpu.PrefetchScalarGridSpec(
        num_scalar_prefetch=2,
        grid=(1,),
        in_specs=[
            pl.BlockSpec(memory_space=pl.ANY),
            pl.BlockSpec(memory_space=pl.ANY),
        ],
        out_specs=pl.BlockSpec(memory_space=pltpu.SMEM),
        scratch_shapes=[
            pltpu.VMEM((NBUF, ROW_SUB, LANE), jnp.float32),
            pltpu.VMEM((NBUF, ROW_SUB, LANE), jnp.float32),
            pltpu.VMEM((8, LANE), jnp.float32),
            pltpu.SemaphoreType.DMA((2, NBUF)),
        ],
    ),
    out_shape=jax.ShapeDtypeStruct((1, 1), jnp.float32),
)


def kernel(recons, x, y):
    perm, nvec = _compact(y)
    r3 = recons.reshape(ROWS, SUB, LANE)
    x3 = x.reshape(ROWS, SUB, LANE)
    out = _loss(perm, nvec, r3, x3)
    return out[0, 0]
